# Initial kernel scaffold; baseline (speedup 1.0000x reference)
import jax
import jax.numpy as jnp
import numpy as np
from jax import lax
from jax.experimental import pallas as pl
from jax.experimental.pallas import tpu as pltpu
from jax.experimental.pallas import tpu_sc as plsc

_SHAPE = (128, 32768)
_TOTAL = _SHAPE[0] * _SHAPE[1]

_MASK_KEY = jax.random.key(42)
_MASK = np.asarray(
    jax.random.uniform(_MASK_KEY, _SHAPE, dtype=jnp.float32) > 0.5
).ravel()
_IDX_NP = np.flatnonzero(_MASK).astype(np.int64)
_N = int(_IDX_NP.shape[0])

_NC, _NS = 2, 16
_NW = _NC * _NS
_N_CHUNKS = 256
_CHUNKS_PER_W = _N_CHUNKS // _NW
_COUT = 8192
assert _N_CHUNKS * _COUT >= _N


def _build_static():
    starts = np.zeros((_N_CHUNKS,), dtype=np.int32)
    locals_ = np.zeros((_N_CHUNKS, _COUT), dtype=np.int64)
    spans = np.zeros((_N_CHUNKS,), dtype=np.int64)
    for c in range(_N_CHUNKS):
        o0 = c * _COUT
        o1 = min(_N, o0 + _COUT)
        chunk = _IDX_NP[o0:o1]
        if chunk.size < _COUT:
            chunk = np.concatenate(
                [chunk, np.full((_COUT - chunk.size,), chunk[-1], np.int64)]
            )
        s = (int(chunk[0]) // 8) * 8
        starts[c] = s
        locals_[c] = chunk - s
        spans[c] = chunk[-1] - s + 1
    w = int(spans.max())
    w = ((w + 15) // 16) * 16
    over = starts > _TOTAL - w
    locals_[over] += (starts[over] - (_TOTAL - w))[:, None]
    starts[over] = _TOTAL - w
    assert locals_.min() >= 0 and locals_.max() < w <= 65536
    blocks = locals_.reshape(_N_CHUNKS, _COUT // 32, 32).astype(np.uint32)
    words = (blocks[:, :, :16] | (blocks[:, :, 16:] << np.uint32(16)))
    words = words.reshape(_N_CHUNKS, _COUT // 2).view(np.int32)
    return words, starts, w


_WORDS, _STARTS, _W = _build_static()


def _sc_body(x_hbm, w_hbm, s_hbm, out_hbm, x_buf, w_buf, o_buf, s_buf):
    wid = lax.axis_index("s") * _NC + lax.axis_index("c")
    pltpu.sync_copy(s_hbm, s_buf)
    for j in range(_CHUNKS_PER_W):
        c = wid * _CHUNKS_PER_W + j
        st = pl.multiple_of(s_buf[c], 8)
        pltpu.sync_copy(x_hbm.at[pl.ds(st, _W)], x_buf)
        pltpu.sync_copy(w_hbm.at[c], w_buf)

        def inner(i, _):
            v = w_buf[pl.ds(i * 16, 16)]
            lo = jnp.bitwise_and(v, jnp.int32(0xFFFF))
            hi = lax.shift_right_logical(v, 16)
            o_buf[pl.ds(i * 32, 16)] = plsc.load_gather(x_buf, [lo])
            o_buf[pl.ds(i * 32 + 16, 16)] = plsc.load_gather(x_buf, [hi])
            return 0

        lax.fori_loop(0, _COUT // 32, inner, 0)
        pltpu.sync_copy(o_buf, out_hbm.at[pl.ds(c * _COUT, _COUT)])


_sc_gather = pl.kernel(
    _sc_body,
    out_type=jax.ShapeDtypeStruct((_N_CHUNKS * _COUT,), jnp.float32),
    mesh=plsc.VectorSubcoreMesh(
        core_axis_name="c", subcore_axis_name="s",
        num_cores=_NC, num_subcores=_NS,
    ),
    scratch_types=[
        pltpu.VMEM((_W,), jnp.float32),
        pltpu.VMEM((_COUT // 2,), jnp.int32),
        pltpu.VMEM((_COUT,), jnp.float32),
        pltpu.VMEM((_N_CHUNKS,), jnp.int32),
    ],
)


@jax.jit
def kernel(x):
    out = _sc_gather(x.reshape(-1), jnp.asarray(_WORDS), jnp.asarray(_STARTS))
    return out[:_N]


# speedup vs baseline: 990.5616x; 990.5616x over previous
import functools

import jax
import jax.numpy as jnp
import numpy as np
from jax import lax
from jax.experimental import pallas as pl
from jax.experimental.pallas import tpu as pltpu
from jax.experimental.pallas import tpu_sc as plsc

_SHAPE = (128, 32768)
_TOTAL = _SHAPE[0] * _SHAPE[1]

_MASK_KEY = jax.random.key(42)
_MASK = np.asarray(
    jax.random.uniform(_MASK_KEY, _SHAPE, dtype=jnp.float32) > 0.5
).ravel()
_IDX_NP = np.flatnonzero(_MASK).astype(np.int64)
_N = int(_IDX_NP.shape[0])

_NC, _NS = 2, 16
_NW = _NC * _NS
_N_CHUNKS = 256
_CHUNKS_PER_W = _N_CHUNKS // _NW
_COUT = 8192
assert _N_CHUNKS * _COUT >= _N


def _build_static():
    starts = np.zeros((_N_CHUNKS,), dtype=np.int32)
    locals_ = np.zeros((_N_CHUNKS, _COUT), dtype=np.int64)
    spans = np.zeros((_N_CHUNKS,), dtype=np.int64)
    for c in range(_N_CHUNKS):
        o0 = c * _COUT
        o1 = min(_N, o0 + _COUT)
        chunk = _IDX_NP[o0:o1]
        if chunk.size < _COUT:
            chunk = np.concatenate(
                [chunk, np.full((_COUT - chunk.size,), chunk[-1], np.int64)]
            )
        s = (int(chunk[0]) // 8) * 8
        starts[c] = s
        locals_[c] = chunk - s
        spans[c] = chunk[-1] - s + 1
    w = int(spans.max())
    w = ((w + 15) // 16) * 16
    over = starts > _TOTAL - w
    locals_[over] += (starts[over] - (_TOTAL - w))[:, None]
    starts[over] = _TOTAL - w
    assert locals_.min() >= 0 and locals_.max() < w <= 65536
    blocks = locals_.reshape(_N_CHUNKS, _COUT // 32, 32).astype(np.uint32)
    words = (blocks[:, :, :16] | (blocks[:, :, 16:] << np.uint32(16)))
    words = words.reshape(_N_CHUNKS, _COUT // 2).view(np.int32)
    starts = np.concatenate([starts, np.zeros((16,), np.int32)])
    return words, starts, w


_WORDS, _STARTS, _W = _build_static()


def _sc_body(x_hbm, w_hbm, s_hbm, out_hbm, x_buf, w_buf, o_buf, s_buf):
    wid = lax.axis_index("s") * _NC + lax.axis_index("c")
    pltpu.sync_copy(s_hbm, s_buf)
    sv = s_buf[pl.ds(wid * _CHUNKS_PER_W, 16)]
    for j in range(_CHUNKS_PER_W):
        c = wid * _CHUNKS_PER_W + j
        st = pl.multiple_of(sv[j], 8)
        pltpu.sync_copy(x_hbm.at[pl.ds(st, _W)], x_buf)
        pltpu.sync_copy(w_hbm.at[c], w_buf)

        def inner(i, _):
            v = w_buf[pl.ds(i * 16, 16)]
            lo = jnp.bitwise_and(v, jnp.int32(0xFFFF))
            hi = lax.shift_right_logical(v, 16)
            o_buf[pl.ds(i * 32, 16)] = plsc.load_gather(x_buf, [lo])
            o_buf[pl.ds(i * 32 + 16, 16)] = plsc.load_gather(x_buf, [hi])
            return 0

        lax.fori_loop(0, _COUT // 32, inner, 0)
        pltpu.sync_copy(o_buf, out_hbm.at[pl.ds(c * _COUT, _COUT)])


@functools.cache
def _sc_gather():
    return pl.kernel(
        _sc_body,
        out_type=jax.ShapeDtypeStruct((_N_CHUNKS * _COUT,), jnp.float32),
        mesh=plsc.VectorSubcoreMesh(
            core_axis_name="c", subcore_axis_name="s",
            num_cores=_NC, num_subcores=_NS,
        ),
        scratch_types=[
            pltpu.VMEM((_W,), jnp.float32),
            pltpu.VMEM((_COUT // 2,), jnp.int32),
            pltpu.VMEM((_COUT,), jnp.float32),
            pltpu.VMEM((_N_CHUNKS + 16,), jnp.int32),
        ],
        compiler_params=pltpu.CompilerParams(
            use_tc_tiling_on_sc=False,
            needs_layout_passes=False,
        ),
    )


@jax.jit
def kernel(x):
    out = _sc_gather()(x.reshape(-1), jnp.asarray(_WORDS), jnp.asarray(_STARTS))
    return out[:_N]


# speedup vs baseline: 1266.1312x; 1.2782x over previous
import functools

import jax
import jax.numpy as jnp
import numpy as np
from jax import lax
from jax.experimental import pallas as pl
from jax.experimental.pallas import tpu as pltpu
from jax.experimental.pallas import tpu_sc as plsc

_SHAPE = (128, 32768)
_TOTAL = _SHAPE[0] * _SHAPE[1]

_MASK_KEY = jax.random.key(42)
_MASK = np.asarray(
    jax.random.uniform(_MASK_KEY, _SHAPE, dtype=jnp.float32) > 0.5
).ravel()
_IDX_NP = np.flatnonzero(_MASK).astype(np.int64)
_N = int(_IDX_NP.shape[0])

_NC, _NS = 2, 16
_NW = _NC * _NS
_N_CHUNKS = 256
_CHUNKS_PER_W = _N_CHUNKS // _NW
_COUT = 8192
assert _N_CHUNKS * _COUT >= _N


def _build_static():
    starts = np.zeros((_N_CHUNKS,), dtype=np.int32)
    locals_ = np.zeros((_N_CHUNKS, _COUT), dtype=np.int64)
    spans = np.zeros((_N_CHUNKS,), dtype=np.int64)
    for c in range(_N_CHUNKS):
        o0 = c * _COUT
        o1 = min(_N, o0 + _COUT)
        chunk = _IDX_NP[o0:o1]
        if chunk.size < _COUT:
            chunk = np.concatenate(
                [chunk, np.full((_COUT - chunk.size,), chunk[-1], np.int64)]
            )
        s = (int(chunk[0]) // 8) * 8
        starts[c] = s
        locals_[c] = chunk - s
        spans[c] = chunk[-1] - s + 1
    w = int(spans.max())
    w = ((w + 15) // 16) * 16
    over = starts > _TOTAL - w
    locals_[over] += (starts[over] - (_TOTAL - w))[:, None]
    starts[over] = _TOTAL - w
    assert locals_.min() >= 0 and locals_.max() < w <= 65536
    blocks = locals_.reshape(_N_CHUNKS, _COUT // 32, 32).astype(np.uint32)
    words = (blocks[:, :, :16] | (blocks[:, :, 16:] << np.uint32(16)))
    words = words.reshape(_N_CHUNKS, _COUT // 2).view(np.int32)
    starts = np.concatenate([starts, np.zeros((16,), np.int32)])
    return words, starts, w


_WORDS, _STARTS, _W = _build_static()


_TAIL = _N - (_N_CHUNKS - 1) * _COUT


def _sc_body(x_hbm, w_hbm, s_hbm, out_hbm, x_buf, w_buf, o_buf, s_buf,
             semx, semw, semo):
    wid = lax.axis_index("s") * _NC + lax.axis_index("c")
    pltpu.sync_copy(s_hbm, s_buf)
    sv = s_buf[pl.ds(wid * _CHUNKS_PER_W, 16)]

    def in_copies(j):
        b = j & 1
        c = wid * _CHUNKS_PER_W + j
        st = pl.multiple_of(sv[j], 8)
        hx = pltpu.make_async_copy(
            x_hbm.at[pl.ds(st, _W)], x_buf.at[b], semx.at[b])
        hw = pltpu.make_async_copy(w_hbm.at[c], w_buf.at[b], semw.at[b])
        hx.start()
        hw.start()
        return hx, hw

    in_h = [None, None]
    out_h = [None, None]
    in_h[0] = in_copies(0)
    for j in range(_CHUNKS_PER_W):
        b = j & 1
        c = wid * _CHUNKS_PER_W + j
        if j + 1 < _CHUNKS_PER_W:
            in_h[(j + 1) & 1] = in_copies(j + 1)
        hx, hw = in_h[b]
        hx.wait()
        hw.wait()
        if out_h[b] is not None:
            out_h[b].wait()
            out_h[b] = None
        xb, wb, ob = x_buf.at[b], w_buf.at[b], o_buf.at[b]

        def inner(i, _):
            v = wb[pl.ds(i * 16, 16)]
            lo = jnp.bitwise_and(v, jnp.int32(0xFFFF))
            hi = lax.shift_right_logical(v, 16)
            ob[pl.ds(i * 32, 16)] = plsc.load_gather(xb, [lo])
            ob[pl.ds(i * 32 + 16, 16)] = plsc.load_gather(xb, [hi])
            return 0

        lax.fori_loop(0, _COUT // 32, inner, 0, unroll=4)
        if j + 1 < _CHUNKS_PER_W:
            ho = pltpu.make_async_copy(
                ob, out_hbm.at[pl.ds(c * _COUT, _COUT)], semo.at[b])
            ho.start()
            out_h[b] = ho
        else:
            @pl.when(wid != _NW - 1)
            def _():
                pltpu.sync_copy(ob, out_hbm.at[pl.ds(c * _COUT, _COUT)])

            @pl.when(wid == _NW - 1)
            def _():
                pltpu.sync_copy(
                    ob.at[pl.ds(0, _TAIL)],
                    out_hbm.at[pl.ds(c * _COUT, _TAIL)])

    for h in out_h:
        if h is not None:
            h.wait()


@functools.cache
def _sc_gather():
    return pl.kernel(
        _sc_body,
        out_type=jax.ShapeDtypeStruct((_N,), jnp.float32),
        mesh=plsc.VectorSubcoreMesh(
            core_axis_name="c", subcore_axis_name="s",
            num_cores=_NC, num_subcores=_NS,
        ),
        scratch_types=[
            pltpu.VMEM((2, _W), jnp.float32),
            pltpu.VMEM((2, _COUT // 2), jnp.int32),
            pltpu.VMEM((2, _COUT), jnp.float32),
            pltpu.VMEM((_N_CHUNKS + 16,), jnp.int32),
            pltpu.SemaphoreType.DMA((2,)),
            pltpu.SemaphoreType.DMA((2,)),
            pltpu.SemaphoreType.DMA((2,)),
        ],
        compiler_params=pltpu.CompilerParams(
            use_tc_tiling_on_sc=False,
            needs_layout_passes=False,
        ),
    )


@jax.jit
def kernel(x):
    return _sc_gather()(x.reshape(-1), jnp.asarray(_WORDS), jnp.asarray(_STARTS))


# speedup vs baseline: 1703.0856x; 1.3451x over previous
import functools

import jax
import jax.numpy as jnp
import numpy as np
from jax import lax
from jax.experimental import pallas as pl
from jax.experimental.pallas import tpu as pltpu
from jax.experimental.pallas import tpu_sc as plsc

_SHAPE = (128, 32768)
_TOTAL = _SHAPE[0] * _SHAPE[1]

_MASK_KEY = jax.random.key(42)
_MASK = np.asarray(
    jax.random.uniform(_MASK_KEY, _SHAPE, dtype=jnp.float32) > 0.5
).ravel()
_IDX_NP = np.flatnonzero(_MASK).astype(np.int64)
_N = int(_IDX_NP.shape[0])

_NC, _NS = 2, 16
_NW = _NC * _NS
_N_CHUNKS = 256
_CHUNKS_PER_W = _N_CHUNKS // _NW
_COUT = 8192
assert _N_CHUNKS * _COUT >= _N


def _build_static():
    starts = np.zeros((_N_CHUNKS,), dtype=np.int32)
    locals_ = np.zeros((_N_CHUNKS, _COUT), dtype=np.int64)
    spans = np.zeros((_N_CHUNKS,), dtype=np.int64)
    for c in range(_N_CHUNKS):
        o0 = c * _COUT
        o1 = min(_N, o0 + _COUT)
        chunk = _IDX_NP[o0:o1]
        if chunk.size < _COUT:
            chunk = np.concatenate(
                [chunk, np.full((_COUT - chunk.size,), chunk[-1], np.int64)]
            )
        s = (int(chunk[0]) // 8) * 8
        starts[c] = s
        locals_[c] = chunk - s
        spans[c] = chunk[-1] - s + 1
    w = int(spans.max())
    w = ((w + 15) // 16) * 16
    over = starts > _TOTAL - w
    locals_[over] += (starts[over] - (_TOTAL - w))[:, None]
    starts[over] = _TOTAL - w
    assert locals_.min() >= 0 and locals_.max() < w <= 65536
    blocks = locals_.reshape(_N_CHUNKS, _COUT // 32, 32).astype(np.uint32)
    words = (blocks[:, :, :16] | (blocks[:, :, 16:] << np.uint32(16)))
    words = words.reshape(_N_CHUNKS, _COUT // 2).view(np.int32)
    starts = np.concatenate([starts, np.zeros((16,), np.int32)])
    return words, starts, w


_WORDS, _STARTS, _W = _build_static()


_TAIL = _N - (_N_CHUNKS - 1) * _COUT


def _sc_body(x_hbm, w_hbm, s_hbm, out_hbm, x_buf, w_buf, o_buf, s_buf,
             semx, semw, semo):
    wid = lax.axis_index("s") * _NC + lax.axis_index("c")
    pltpu.sync_copy(s_hbm, s_buf)
    sv = s_buf[pl.ds(wid * _CHUNKS_PER_W, 16)]

    def in_copies(j):
        b = j & 1
        c = wid * _CHUNKS_PER_W + j
        st = pl.multiple_of(sv[j], 8)
        hx = pltpu.make_async_copy(
            x_hbm.at[pl.ds(st, _W)], x_buf.at[b], semx.at[b])
        hw = pltpu.make_async_copy(w_hbm.at[c], w_buf.at[b], semw.at[b])
        hx.start()
        hw.start()
        return hx, hw

    in_h = [None, None]
    out_h = [None, None]
    in_h[0] = in_copies(0)
    for j in range(_CHUNKS_PER_W):
        b = j & 1
        c = wid * _CHUNKS_PER_W + j
        if j + 1 < _CHUNKS_PER_W:
            in_h[(j + 1) & 1] = in_copies(j + 1)
        hx, hw = in_h[b]
        hx.wait()
        hw.wait()
        if out_h[b] is not None:
            out_h[b].wait()
            out_h[b] = None
        xb, wb, ob = x_buf.at[b], w_buf.at[b], o_buf.at[b]

        @plsc.parallel_loop(0, _COUT // 32, unroll=8)
        def _(i):
            v = wb[pl.ds(i * 16, 16)]
            lo = jnp.bitwise_and(v, jnp.int32(0xFFFF))
            hi = lax.shift_right_logical(v, 16)
            ob[pl.ds(i * 32, 16)] = plsc.load_gather(xb, [lo])
            ob[pl.ds(i * 32 + 16, 16)] = plsc.load_gather(xb, [hi])
        if j + 1 < _CHUNKS_PER_W:
            ho = pltpu.make_async_copy(
                ob, out_hbm.at[pl.ds(c * _COUT, _COUT)], semo.at[b])
            ho.start()
            out_h[b] = ho
        else:
            @pl.when(wid != _NW - 1)
            def _():
                pltpu.sync_copy(ob, out_hbm.at[pl.ds(c * _COUT, _COUT)])

            @pl.when(wid == _NW - 1)
            def _():
                pltpu.sync_copy(
                    ob.at[pl.ds(0, _TAIL)],
                    out_hbm.at[pl.ds(c * _COUT, _TAIL)])

    for h in out_h:
        if h is not None:
            h.wait()


@functools.cache
def _sc_gather():
    return pl.kernel(
        _sc_body,
        out_type=jax.ShapeDtypeStruct((_N,), jnp.float32),
        mesh=plsc.VectorSubcoreMesh(
            core_axis_name="c", subcore_axis_name="s",
            num_cores=_NC, num_subcores=_NS,
        ),
        scratch_types=[
            pltpu.VMEM((2, _W), jnp.float32),
            pltpu.VMEM((2, _COUT // 2), jnp.int32),
            pltpu.VMEM((2, _COUT), jnp.float32),
            pltpu.VMEM((_N_CHUNKS + 16,), jnp.int32),
            pltpu.SemaphoreType.DMA((2,)),
            pltpu.SemaphoreType.DMA((2,)),
            pltpu.SemaphoreType.DMA((2,)),
        ],
        compiler_params=pltpu.CompilerParams(
            use_tc_tiling_on_sc=False,
            needs_layout_passes=False,
        ),
    )


@jax.jit
def kernel(x):
    return _sc_gather()(x.reshape(-1), jnp.asarray(_WORDS), jnp.asarray(_STARTS))
